# baseline (speedup 1.0000x reference)
import functools

import jax
import jax.numpy as jnp
from jax.experimental import pallas as pl
from jax.experimental.pallas import tpu as pltpu

NUM_HEADS = 16
HEAD_DIM = 128
_WB = 256


def _cast_kernel(wq_ref, wk_ref, wv_ref, wo_ref, wc_ref, wob_ref):
    wc_ref[0:_WB] = wq_ref[...].astype(jnp.bfloat16)
    wc_ref[_WB:2 * _WB] = wk_ref[...].astype(jnp.bfloat16)
    wc_ref[2 * _WB:3 * _WB] = wv_ref[...].astype(jnp.bfloat16)
    wob_ref[...] = wo_ref[...].astype(jnp.bfloat16)


def _cast_weights(wq, wk, wv, wo, interpret=False):
    n, k = wq.shape
    in_spec = pl.BlockSpec((_WB, k), lambda i: (i, 0))
    wc_spec = pl.BlockSpec((3 * _WB, k), lambda i: (i, 0))
    return pl.pallas_call(
        _cast_kernel, grid=(n // _WB,),
        in_specs=[in_spec] * 4,
        out_specs=[wc_spec, in_spec],
        out_shape=[jax.ShapeDtypeStruct((3 * n, k), jnp.bfloat16),
                   jax.ShapeDtypeStruct((n, k), jnp.bfloat16)],
        interpret=interpret)(wq, wk, wv, wo)


def _qkv_kernel(x_ref, wc_ref, o_ref, *, scale, n3):
    xb = x_ref[...].astype(jnp.bfloat16)
    acc = jax.lax.dot_general(
        xb, wc_ref[...], (((1,), (1,)), ((), ())),
        preferred_element_type=jnp.float32)
    for t in range(n3 // (3 * _WB)):
        g = t * 3 * _WB
        o_ref[:, g:g + _WB] = (
            acc[:, g:g + _WB] * scale).astype(jnp.bfloat16)
        o_ref[:, g + _WB:g + 3 * _WB] = (
            acc[:, g + _WB:g + 3 * _WB]).astype(jnp.bfloat16)


def _qkv_proj(x2, wc, bm, interpret=False):
    m, k = x2.shape
    n3 = wc.shape[0]
    scale = 1.0 / (HEAD_DIM ** 0.5)
    x_spec = pl.BlockSpec((bm, k), lambda i: (i, 0))
    w_spec = pl.BlockSpec((n3, k), lambda i: (0, 0))
    o_spec = pl.BlockSpec((bm, n3), lambda i: (i, 0))
    return pl.pallas_call(
        functools.partial(_qkv_kernel, scale=scale, n3=n3), grid=(m // bm,),
        in_specs=[x_spec, w_spec],
        out_specs=o_spec,
        out_shape=jax.ShapeDtypeStruct((m, n3), jnp.bfloat16),
        interpret=interpret)(x2, wc)


def _out_kernel(a_ref, w_ref, b_ref, o_ref):
    acc = jax.lax.dot_general(
        a_ref[...], w_ref[...], (((1,), (1,)), ((), ())),
        preferred_element_type=jnp.float32)
    o_ref[...] = acc + b_ref[...]


def _out_proj(attn2, wob, bo, bm, interpret=False):
    m, k = attn2.shape
    n = wob.shape[0]
    a_spec = pl.BlockSpec((bm, k), lambda i: (i, 0))
    w_spec = pl.BlockSpec((n, k), lambda i: (0, 0))
    b_spec = pl.BlockSpec((1, n), lambda i: (0, 0))
    o_spec = pl.BlockSpec((bm, n), lambda i: (i, 0))
    return pl.pallas_call(
        _out_kernel, grid=(m // bm,),
        in_specs=[a_spec, w_spec, b_spec],
        out_specs=o_spec,
        out_shape=jax.ShapeDtypeStruct((m, n), jnp.float32),
        interpret=interpret)(attn2, wob, bo.reshape(1, n))


def _flash_kernel(q_ref, k_ref, v_ref, o_ref, s_scr, *, bq, bk, nq):
    qi = pl.program_id(1)
    q = q_ref[0]
    nlanes = 128
    ncol = bk // nlanes
    rows = jax.lax.broadcasted_iota(jnp.int32, (bq, bk), 0)
    cols = jax.lax.broadcasted_iota(jnp.int32, (bq, bk), 1)
    ones = jnp.ones((bk, HEAD_DIM), jnp.bfloat16)

    def fold_max(macc, s):
        nr = s.shape[0]
        for c in range(ncol):
            macc = jnp.maximum(
                macc, jax.lax.slice(s, (0, c * nlanes), (nr, (c + 1) * nlanes)))
        return macc

    for sqi in range(nq):
        @pl.when(qi == sqi)
        def _(sqi=sqi):
            nkb = (sqi + 1) * bq // bk
            macc = jnp.full((bq, nlanes), -jnp.inf, jnp.float32)
            for j in range(nkb):
                kb = k_ref[0, j * bk:(j + 1) * bk, :]
                s = jax.lax.dot_general(
                    q, kb, dimension_numbers=(((1,), (1,)), ((), ())),
                    preferred_element_type=jnp.float32)
                off = sqi * bq - j * bk
                if off < bk:
                    s = jnp.where(cols <= rows + off, s, -jnp.inf)
                s_scr[:, j * bk:(j + 1) * bk] = s
                macc = fold_max(macc, s)
            m = jnp.max(macc, axis=1, keepdims=True)
            lacc = jnp.zeros((bq, HEAD_DIM), jnp.float32)
            acc = jnp.zeros((bq, HEAD_DIM), jnp.float32)
            for j in range(nkb):
                p = jnp.exp(s_scr[:, j * bk:(j + 1) * bk] - m
                            ).astype(jnp.bfloat16)
                lacc = lacc + jnp.dot(p, ones,
                                      preferred_element_type=jnp.float32)
                vb = v_ref[0, j * bk:(j + 1) * bk, :]
                acc = acc + jnp.dot(p, vb, preferred_element_type=jnp.float32)
            o_ref[0] = (acc / lacc).astype(o_ref.dtype)


def _flash_attention(qkv3, bq, bk, interpret=False):
    b, s, n3 = qkv3.shape
    hidden = n3 // 3
    grid = (b * NUM_HEADS, s // bq)

    def _idx(off):
        def im(bh, qi_, _off=off):
            h = bh % NUM_HEADS
            return (bh // NUM_HEADS, 0, 6 * (h // 2) + (h % 2) + _off)
        return im

    def _q_idx(bh, qi_):
        h = bh % NUM_HEADS
        return (bh // NUM_HEADS, qi_, 6 * (h // 2) + (h % 2))

    q_spec = pl.BlockSpec((1, bq, HEAD_DIM), _q_idx)
    k_spec = pl.BlockSpec((1, s, HEAD_DIM), _idx(2))
    v_spec = pl.BlockSpec((1, s, HEAD_DIM), _idx(4))
    o_spec = pl.BlockSpec(
        (1, bq, HEAD_DIM),
        lambda bh, qi_: (bh // NUM_HEADS, qi_, bh % NUM_HEADS))
    return pl.pallas_call(
        functools.partial(_flash_kernel, bq=bq, bk=bk, nq=s // bq),
        grid=grid,
        in_specs=[q_spec, k_spec, v_spec],
        out_specs=o_spec,
        out_shape=jax.ShapeDtypeStruct((b, s, hidden), jnp.bfloat16),
        scratch_shapes=[pltpu.VMEM((bq, s), jnp.float32)],
        interpret=interpret)(qkv3, qkv3, qkv3)


def kernel(x, Wq, Wk, Wv, Wo, bo, interpret=False):
    b, s, hidden = x.shape
    wc, wob = _cast_weights(Wq, Wk, Wv, Wo, interpret=interpret)
    x2 = x.reshape(b * s, hidden)
    qkv = _qkv_proj(x2, wc, bm=512, interpret=interpret)
    qkv3 = qkv.reshape(b, s, 3 * hidden)
    attn = _flash_attention(qkv3, bq=1024, bk=1024, interpret=interpret)
    out = _out_proj(attn.reshape(b * s, hidden), wob, bo, bm=512,
                    interpret=interpret)
    return out.reshape(b, s, hidden)
